# baseline (speedup 1.0000x reference)
import functools

import jax
import jax.numpy as jnp
from jax import lax
from jax.experimental import pallas as pl
from jax.experimental.pallas import tpu as pltpu
from jax.experimental.pallas import tpu_sc as plsc

_B = 16
_S = 64
_R = _S - 1
_D = 1024
_NW = 32
_TASKS = _B * _R
_RUNW = _S * _D
_PER_W = (_TASKS + _NW - 1) // _NW


def _run_offsets(t):
    b = t // _R
    i = t - b * _R
    src = (b * (_S * _S) + i * (_S + 1) + 1) * _D
    dst = (b * (_R * _S) + i * _S) * _D
    return src, dst


def kernel(arr):
    B, S2, D = arr.shape
    src1 = arr.reshape(B * S2 * D)

    mesh = plsc.VectorSubcoreMesh(core_axis_name="c", subcore_axis_name="s")

    @functools.partial(
        pl.kernel,
        mesh=mesh,
        out_type=jax.ShapeDtypeStruct((_B * _R * _S * _D,), arr.dtype),
        scratch_types=[pltpu.SemaphoreType.DMA],
    )
    def copy_kernel(in_hbm, out_hbm, sem):
        wid = lax.axis_index("s") * 2 + lax.axis_index("c")

        def fire(t):
            src, dst = _run_offsets(t)
            pltpu.async_copy(
                in_hbm.at[pl.ds(src, _RUNW)], out_hbm.at[pl.ds(dst, _RUNW)], sem
            )

        def drain():
            pltpu.make_async_copy(
                in_hbm.at[pl.ds(0, _RUNW)], out_hbm.at[pl.ds(0, _RUNW)], sem
            ).wait()

        for j in range(_PER_W - 1):
            fire(wid + j * _NW)
        last = wid + (_PER_W - 1) * _NW

        @pl.when(last < _TASKS)
        def _():
            fire(last)

        for _ in range(_PER_W - 1):
            drain()

        @pl.when(last < _TASKS)
        def _():
            drain()

    out1 = copy_kernel(src1)
    return out1.reshape(B, _R * _S, D)


# speedup vs baseline: 12.7099x; 12.7099x over previous
import functools

import jax
import jax.numpy as jnp
from jax import lax
from jax.experimental import pallas as pl
from jax.experimental.pallas import tpu as pltpu
from jax.experimental.pallas import tpu_sc as plsc

_B = 16
_S = 64
_R = _S - 1
_D = 1024
_CH = 32
_NW = 32
_NBUF = 3
_CHUNKS = _B * _R * 2
_PER_W = _CHUNKS // _NW
_CHW = _CH * _D


def _chunk_offs(g):
    task = g // 2
    half = g - task * 2
    b = task // _R
    i = task - b * _R
    src = b * (_S * _S) + i * (_S + 1) + 1 + half * _CH
    dst = b * (_R * _S) + i * _S + half * _CH
    return src * _D, dst * _D


def kernel(arr):
    B, S2, D = arr.shape
    src1 = arr.reshape(B * S2 * D)

    mesh = plsc.VectorSubcoreMesh(core_axis_name="c", subcore_axis_name="s")

    @functools.partial(
        pl.kernel,
        mesh=mesh,
        out_type=jax.ShapeDtypeStruct((_B * _R * _S * _D,), arr.dtype),
        scratch_types=(
            [pltpu.VMEM((_CHW,), jnp.float32) for _ in range(_NBUF)]
            + [pltpu.SemaphoreType.DMA for _ in range(2 * _NBUF)]
        ),
    )
    def copy_kernel(in_hbm, out_hbm, *rest):
        bufs = rest[:_NBUF]
        gsems = rest[_NBUF : 2 * _NBUF]
        ssems = rest[2 * _NBUF :]
        wid = lax.axis_index("s") * 2 + lax.axis_index("c")

        def gather(j, ph):
            src, _ = _chunk_offs(wid + j * _NW)
            pltpu.async_copy(in_hbm.at[pl.ds(src, _CHW)], bufs[ph], gsems[ph])

        def scatter(j, ph):
            _, dst = _chunk_offs(wid + j * _NW)
            pltpu.async_copy(bufs[ph], out_hbm.at[pl.ds(dst, _CHW)], ssems[ph])

        def wait_gather(ph):
            pltpu.make_async_copy(
                in_hbm.at[pl.ds(0, _CHW)], bufs[ph], gsems[ph]
            ).wait()

        def wait_scatter(ph):
            pltpu.make_async_copy(
                bufs[ph], out_hbm.at[pl.ds(0, _CHW)], ssems[ph]
            ).wait()

        for j in range(_NBUF):
            gather(j, j % _NBUF)
        for j in range(_PER_W):
            ph = j % _NBUF
            wait_gather(ph)
            scatter(j, ph)
            if j + _NBUF < _PER_W:
                wait_scatter(ph)
                gather(j + _NBUF, ph)
        for j in range(_PER_W - min(_NBUF, _PER_W), _PER_W):
            wait_scatter(j % _NBUF)

    out1 = copy_kernel(src1)
    return out1.reshape(B, _R * _S, D)


# speedup vs baseline: 13.1452x; 1.0342x over previous
import functools

import jax
import jax.numpy as jnp
from jax import lax
from jax.experimental import pallas as pl
from jax.experimental.pallas import tpu as pltpu
from jax.experimental.pallas import tpu_sc as plsc

_B = 16
_S = 64
_R = _S - 1
_D = 1024
_CH = 32
_NW = 32
_NBUF = 3
_CHUNKS = _B * _R * 2
_PER_W = _CHUNKS // _NW
_CHW = _CH * _D


def _chunk_offs(g):
    task = g // 2
    half = g - task * 2
    b = task // _R
    i = task - b * _R
    src = b * (_S * _S) + i * (_S + 1) + 1 + half * _CH
    dst = b * (_R * _S) + i * _S + half * _CH
    return src * _D, dst * _D


def kernel(arr):
    B, S2, D = arr.shape
    src1 = arr.reshape(B * S2 * D)

    mesh = plsc.VectorSubcoreMesh(core_axis_name="c", subcore_axis_name="s")

    @functools.partial(
        pl.kernel,
        mesh=mesh,
        out_type=jax.ShapeDtypeStruct((_B * _R * _S * _D,), arr.dtype),
        scratch_types=(
            [pltpu.VMEM_SHARED((16 * _NBUF * _CHW,), jnp.float32)]
            + [pltpu.SemaphoreType.DMA for _ in range(2 * _NBUF)]
        ),
    )
    def copy_kernel(in_hbm, out_hbm, shared, *rest):
        gsems = rest[:_NBUF]
        ssems = rest[_NBUF :]
        sid = lax.axis_index("s")
        wid = sid * 2 + lax.axis_index("c")

        def gather(j, ph):
            src, _ = _chunk_offs(wid + j * _NW)
            pltpu.async_copy(
                in_hbm.at[pl.ds(src, _CHW)], shared.at[pl.ds((sid * _NBUF + ph) * _CHW, _CHW)], gsems[ph]
            )

        def scatter(j, ph):
            _, dst = _chunk_offs(wid + j * _NW)
            pltpu.async_copy(
                shared.at[pl.ds((sid * _NBUF + ph) * _CHW, _CHW)], out_hbm.at[pl.ds(dst, _CHW)], ssems[ph]
            )

        def wait_gather(ph):
            pltpu.make_async_copy(
                in_hbm.at[pl.ds(0, _CHW)], shared.at[pl.ds((sid * _NBUF + ph) * _CHW, _CHW)], gsems[ph]
            ).wait()

        def wait_scatter(ph):
            pltpu.make_async_copy(
                shared.at[pl.ds((sid * _NBUF + ph) * _CHW, _CHW)], out_hbm.at[pl.ds(0, _CHW)], ssems[ph]
            ).wait()

        for j in range(_NBUF):
            gather(j, j % _NBUF)
        for j in range(_PER_W):
            ph = j % _NBUF
            wait_gather(ph)
            scatter(j, ph)
            if j + _NBUF < _PER_W:
                wait_scatter(ph)
                gather(j + _NBUF, ph)
        for j in range(_PER_W - min(_NBUF, _PER_W), _PER_W):
            wait_scatter(j % _NBUF)

    out1 = copy_kernel(src1)
    return out1.reshape(B, _R * _S, D)
